# baseline (speedup 1.0000x reference)
import jax
import jax.numpy as jnp
from jax import lax
from jax.experimental import pallas as pl
from jax.experimental.pallas import tpu as pltpu
from jax.experimental.pallas import tpu_sc as plsc

_D = 128
_L = 16
_NC = 2
_NS = 16
_NW = _NC * _NS
_Q = 10.0
_C = 160


def _sc_scores_body(table, nidx, pidx, negidx,
                    pos_out, neg_out,
                    nidx_v, pidx_v, negidx_v, negrows_v, vvec,
                    nbuf0, pbuf0, nbuf1, pbuf1,
                    pscore_v, nscore_v,
                    semn0, semp0, semn1, semp1, semneg):
    s_total = pos_out.shape[0]
    rpw = s_total // _NW
    nch = rpw // _C
    neg = negidx_v.shape[0]

    cid = lax.axis_index("c")
    sid = lax.axis_index("s")
    wid = sid * _NC + cid
    base = wid * rpw

    pltpu.sync_copy(nidx.at[pl.ds(base, rpw)], nidx_v)
    pltpu.sync_copy(pidx.at[pl.ds(base, rpw)], pidx_v)
    pltpu.sync_copy(negidx, negidx_v)

    iota = lax.iota(jnp.int32, _L)

    _H = _C // 2

    def start(c, nbuf, pbuf, semn, semp):
        pltpu.async_copy(table.at[nidx_v.at[pl.ds(c * _C, _H)]],
                         nbuf.at[pl.ds(0, _H)], semn)
        pltpu.async_copy(table.at[pidx_v.at[pl.ds(c * _C, _H)]],
                         pbuf.at[pl.ds(0, _H)], semp)
        pltpu.async_copy(table.at[nidx_v.at[pl.ds(c * _C + _H, _H)]],
                         nbuf.at[pl.ds(_H, _H)], semn)
        pltpu.async_copy(table.at[pidx_v.at[pl.ds(c * _C + _H, _H)]],
                         pbuf.at[pl.ds(_H, _H)], semp)

    def wait(c, nbuf, pbuf, semn, semp):
        pltpu.make_async_copy(
            table.at[nidx_v.at[pl.ds(c * _C, _H)]],
            nbuf.at[pl.ds(0, _H)], semn).wait()
        pltpu.make_async_copy(
            table.at[pidx_v.at[pl.ds(c * _C, _H)]],
            pbuf.at[pl.ds(0, _H)], semp).wait()
        pltpu.make_async_copy(
            table.at[nidx_v.at[pl.ds(c * _C + _H, _H)]],
            nbuf.at[pl.ds(_H, _H)], semn).wait()
        pltpu.make_async_copy(
            table.at[pidx_v.at[pl.ds(c * _C + _H, _H)]],
            pbuf.at[pl.ds(_H, _H)], semp).wait()

    bfly = [iota ^ (1 << b) for b in range(4)]
    zero = jnp.zeros((_L,), jnp.float32)

    def lane_sum(x):
        for b in bfly:
            x = x + jnp.take_along_axis(x, b, axis=0)
        return x

    def compute(c, nbuf, pbuf):
        def g_body(g, carry):
            vch = [vvec[pl.ds(dblk * _L, _L)] for dblk in range(_D // _L)]
            scores_p = zero
            scores_n = zero
            for j in range(_L):
                r = g * _L + j
                acc_p0 = zero
                acc_p1 = zero
                acc_n0 = zero
                acc_n1 = zero
                for dblk in range(_D // _L):
                    nt = nbuf[r, pl.ds(dblk * _L, _L)]
                    pt = pbuf[r, pl.ds(dblk * _L, _L)]
                    if dblk % 2 == 0:
                        acc_p0 = acc_p0 + nt * pt
                        acc_n0 = acc_n0 + nt * vch[dblk]
                    else:
                        acc_p1 = acc_p1 + nt * pt
                        acc_n1 = acc_n1 + nt * vch[dblk]
                sel = iota == j
                scores_p = jnp.where(sel, lane_sum(acc_p0 + acc_p1), scores_p)
                scores_n = jnp.where(sel, lane_sum(acc_n0 + acc_n1), scores_n)
            off = c * _C + g * _L
            pscore_v[pl.ds(off, _L)] = scores_p
            nscore_v[pl.ds(off, _L)] = scores_n
            return carry
        lax.fori_loop(0, _C // _L, g_body, 0)

    start(0, nbuf0, pbuf0, semn0, semp0)

    pltpu.async_copy(table.at[negidx_v], negrows_v, semneg).wait()
    for dblk in range(_D // _L):
        acc = jnp.zeros((_L,), jnp.float32)
        for r in range(neg):
            acc = acc + negrows_v[r, pl.ds(dblk * _L, _L)]
        vvec[pl.ds(dblk * _L, _L)] = acc

    def chunk_pair(i, carry):
        c0 = 2 * i
        c1 = c0 + 1
        start(c1, nbuf1, pbuf1, semn1, semp1)
        wait(c0, nbuf0, pbuf0, semn0, semp0)
        compute(c0, nbuf0, pbuf0)

        @pl.when(c0 + 2 < nch)
        def _():
            start(c0 + 2, nbuf0, pbuf0, semn0, semp0)

        wait(c1, nbuf1, pbuf1, semn1, semp1)
        compute(c1, nbuf1, pbuf1)
        return carry

    lax.fori_loop(0, nch // 2, chunk_pair, 0)

    pltpu.async_copy(pscore_v, pos_out.at[pl.ds(base, rpw)], semn0)
    pltpu.async_copy(nscore_v, neg_out.at[pl.ds(base, rpw)], semp0)
    pltpu.make_async_copy(pscore_v, pos_out.at[pl.ds(base, rpw)], semn0).wait()
    pltpu.make_async_copy(nscore_v, neg_out.at[pl.ds(base, rpw)], semp0).wait()


def _loss_body(p_ref, n_ref, o_ref):
    p = p_ref[...]
    n = n_ref[...]
    s = float(p.size)
    sp = jnp.sum(jax.nn.softplus(-p))
    sn = jnp.sum(jax.nn.softplus(n))
    o_ref[...] = (sp / s + _Q * (sn / s)).reshape(1, 1)


@jax.jit
def kernel(node_embedding, node_indices, pos_indices, neg_indices):
    s_total = node_indices.shape[0]
    rpw = s_total // _NW
    neg = neg_indices.shape[0]
    out_t = (jax.ShapeDtypeStruct((s_total,), jnp.float32),
             jax.ShapeDtypeStruct((s_total,), jnp.float32))
    scratch = [
        pltpu.VMEM((rpw,), jnp.int32),
        pltpu.VMEM((rpw,), jnp.int32),
        pltpu.VMEM((neg,), jnp.int32),
        pltpu.VMEM((neg, _D), jnp.float32),
        pltpu.VMEM((_D,), jnp.float32),
        pltpu.VMEM((_C, _D), jnp.float32),
        pltpu.VMEM((_C, _D), jnp.float32),
        pltpu.VMEM((_C, _D), jnp.float32),
        pltpu.VMEM((_C, _D), jnp.float32),
        pltpu.VMEM((rpw,), jnp.float32),
        pltpu.VMEM((rpw,), jnp.float32),
        pltpu.SemaphoreType.DMA,
        pltpu.SemaphoreType.DMA,
        pltpu.SemaphoreType.DMA,
        pltpu.SemaphoreType.DMA,
        pltpu.SemaphoreType.DMA,
    ]
    pos_s, neg_s = pl.kernel(
        _sc_scores_body,
        out_type=out_t,
        mesh=plsc.VectorSubcoreMesh(core_axis_name="c", subcore_axis_name="s"),
        scratch_types=scratch,
        compiler_params=pltpu.CompilerParams(needs_layout_passes=False),
    )(node_embedding, node_indices, pos_indices, neg_indices)
    rows = s_total // _D
    loss = pl.pallas_call(
        _loss_body,
        out_shape=jax.ShapeDtypeStruct((1, 1), jnp.float32),
    )(pos_s.reshape(rows, _D), neg_s.reshape(rows, _D))
    return loss.reshape(1)


# speedup vs baseline: 1.0196x; 1.0196x over previous
import jax
import jax.numpy as jnp
from jax import lax
from jax.experimental import pallas as pl
from jax.experimental.pallas import tpu as pltpu
from jax.experimental.pallas import tpu_sc as plsc

_D = 128
_L = 16
_NC = 2
_NS = 16
_NW = _NC * _NS
_Q = 10.0
_C = 160


def _sc_scores_body(table, nidx, pidx, negidx,
                    pos_out, neg_out,
                    nidx_v, pidx_v, negidx_v, negrows_v, vvec,
                    nbuf0, pbuf0, nbuf1, pbuf1,
                    pscore_v, nscore_v,
                    semn0, semp0, semn1, semp1, semneg):
    s_total = pos_out.shape[0]
    rpw = s_total // _NW
    nch = rpw // _C
    neg = negidx_v.shape[0]

    cid = lax.axis_index("c")
    sid = lax.axis_index("s")
    wid = sid * _NC + cid
    base = wid * rpw

    pltpu.async_copy(nidx.at[pl.ds(base, rpw)], nidx_v, semn0)
    pltpu.async_copy(pidx.at[pl.ds(base, rpw)], pidx_v, semp0)
    pltpu.async_copy(negidx, negidx_v, semneg)
    pltpu.make_async_copy(nidx.at[pl.ds(base, rpw)], nidx_v, semn0).wait()
    pltpu.make_async_copy(pidx.at[pl.ds(base, rpw)], pidx_v, semp0).wait()

    iota = lax.iota(jnp.int32, _L)

    def start(c, nbuf, pbuf, semn, semp):
        pltpu.async_copy(table.at[nidx_v.at[pl.ds(c * _C, _C)]], nbuf, semn)
        pltpu.async_copy(table.at[pidx_v.at[pl.ds(c * _C, _C)]], pbuf, semp)

    def wait(c, nbuf, pbuf, semn, semp):
        pltpu.make_async_copy(
            table.at[nidx_v.at[pl.ds(c * _C, _C)]], nbuf, semn).wait()
        pltpu.make_async_copy(
            table.at[pidx_v.at[pl.ds(c * _C, _C)]], pbuf, semp).wait()

    bfly = [iota ^ (1 << b) for b in range(4)]
    zero = jnp.zeros((_L,), jnp.float32)

    def lane_sum(x):
        for b in bfly:
            x = x + jnp.take_along_axis(x, b, axis=0)
        return x

    def compute(c, nbuf, pbuf):
        def g_body(g, carry):
            vch = [vvec[pl.ds(dblk * _L, _L)] for dblk in range(_D // _L)]
            scores_p = zero
            scores_n = zero
            for j in range(_L):
                r = g * _L + j
                acc_p0 = zero
                acc_p1 = zero
                acc_n0 = zero
                acc_n1 = zero
                for dblk in range(_D // _L):
                    nt = nbuf[r, pl.ds(dblk * _L, _L)]
                    pt = pbuf[r, pl.ds(dblk * _L, _L)]
                    if dblk % 2 == 0:
                        acc_p0 = acc_p0 + nt * pt
                        acc_n0 = acc_n0 + nt * vch[dblk]
                    else:
                        acc_p1 = acc_p1 + nt * pt
                        acc_n1 = acc_n1 + nt * vch[dblk]
                sel = iota == j
                scores_p = jnp.where(sel, lane_sum(acc_p0 + acc_p1), scores_p)
                scores_n = jnp.where(sel, lane_sum(acc_n0 + acc_n1), scores_n)
            off = c * _C + g * _L
            pscore_v[pl.ds(off, _L)] = scores_p
            nscore_v[pl.ds(off, _L)] = scores_n
            return carry
        lax.fori_loop(0, _C // _L, g_body, 0)

    start(0, nbuf0, pbuf0, semn0, semp0)

    pltpu.make_async_copy(negidx, negidx_v, semneg).wait()
    pltpu.async_copy(table.at[negidx_v], negrows_v, semneg).wait()
    for dblk in range(_D // _L):
        acc = jnp.zeros((_L,), jnp.float32)
        for r in range(neg):
            acc = acc + negrows_v[r, pl.ds(dblk * _L, _L)]
        vvec[pl.ds(dblk * _L, _L)] = acc

    def chunk_pair(i, carry):
        c0 = 2 * i
        c1 = c0 + 1
        start(c1, nbuf1, pbuf1, semn1, semp1)
        wait(c0, nbuf0, pbuf0, semn0, semp0)
        compute(c0, nbuf0, pbuf0)

        @pl.when(c0 + 2 < nch)
        def _():
            start(c0 + 2, nbuf0, pbuf0, semn0, semp0)

        wait(c1, nbuf1, pbuf1, semn1, semp1)
        compute(c1, nbuf1, pbuf1)
        return carry

    lax.fori_loop(0, nch // 2, chunk_pair, 0)

    pltpu.async_copy(pscore_v, pos_out.at[pl.ds(base, rpw)], semn0)
    pltpu.async_copy(nscore_v, neg_out.at[pl.ds(base, rpw)], semp0)
    pltpu.make_async_copy(pscore_v, pos_out.at[pl.ds(base, rpw)], semn0).wait()
    pltpu.make_async_copy(nscore_v, neg_out.at[pl.ds(base, rpw)], semp0).wait()


def _loss_body(p_ref, n_ref, o_ref):
    p = p_ref[...]
    n = n_ref[...]
    s = float(p.size)
    sp = jnp.sum(jax.nn.softplus(-p))
    sn = jnp.sum(jax.nn.softplus(n))
    o_ref[...] = (sp / s + _Q * (sn / s)).reshape(1, 1)


@jax.jit
def kernel(node_embedding, node_indices, pos_indices, neg_indices):
    s_total = node_indices.shape[0]
    rpw = s_total // _NW
    neg = neg_indices.shape[0]
    out_t = (jax.ShapeDtypeStruct((s_total,), jnp.float32),
             jax.ShapeDtypeStruct((s_total,), jnp.float32))
    scratch = [
        pltpu.VMEM((rpw,), jnp.int32),
        pltpu.VMEM((rpw,), jnp.int32),
        pltpu.VMEM((neg,), jnp.int32),
        pltpu.VMEM((neg, _D), jnp.float32),
        pltpu.VMEM((_D,), jnp.float32),
        pltpu.VMEM((_C, _D), jnp.float32),
        pltpu.VMEM((_C, _D), jnp.float32),
        pltpu.VMEM((_C, _D), jnp.float32),
        pltpu.VMEM((_C, _D), jnp.float32),
        pltpu.VMEM((rpw,), jnp.float32),
        pltpu.VMEM((rpw,), jnp.float32),
        pltpu.SemaphoreType.DMA,
        pltpu.SemaphoreType.DMA,
        pltpu.SemaphoreType.DMA,
        pltpu.SemaphoreType.DMA,
        pltpu.SemaphoreType.DMA,
    ]
    pos_s, neg_s = pl.kernel(
        _sc_scores_body,
        out_type=out_t,
        mesh=plsc.VectorSubcoreMesh(core_axis_name="c", subcore_axis_name="s"),
        scratch_types=scratch,
        compiler_params=pltpu.CompilerParams(needs_layout_passes=False),
    )(node_embedding, node_indices, pos_indices, neg_indices)
    rows = s_total // _D
    loss = pl.pallas_call(
        _loss_body,
        out_shape=jax.ShapeDtypeStruct((1, 1), jnp.float32),
    )(pos_s.reshape(rows, _D), neg_s.reshape(rows, _D))
    return loss.reshape(1)
